# baseline (speedup 1.0000x reference)
import functools

import jax
import jax.numpy as jnp
from jax import lax
from jax.experimental import pallas as pl
from jax.experimental.pallas import tpu as pltpu
from jax.experimental.pallas import tpu_sc as plsc

NC = 2
NS = 16
NW = NC * NS
LANE = 16

N = 200000
D = 128
E = 500000
E_PAD = 507904
N_PAD = 204800
N_OUT = 200704
CB = 64
DCH = 1984
STRIPE = N_OUT // NW
SUB = 128
HITCAP = 18560
SUBCAP = 2048


def _sc_gather(table, idx, K):
    B = idx.shape[0]
    Dt = table.shape[1]
    chunks = B // (NW * K)
    bw = B // NW
    mesh = plsc.VectorSubcoreMesh(core_axis_name="c", subcore_axis_name="s")

    @functools.partial(
        pl.kernel,
        out_type=jax.ShapeDtypeStruct((B, Dt), jnp.float32),
        mesh=mesh,
        scratch_types=[
            pltpu.VMEM((K,), jnp.int32),
            pltpu.VMEM((K, Dt), jnp.float32),
            pltpu.SemaphoreType.DMA,
        ],
    )
    def k(table_hbm, idx_hbm, out_hbm, idx_v, rows_v, sem):
        wid = lax.axis_index("s") * NC + lax.axis_index("c")
        base = wid * bw

        def body(i, carry):
            off = base + i * K
            pltpu.sync_copy(idx_hbm.at[pl.ds(off, K)], idx_v)
            pltpu.async_copy(table_hbm.at[idx_v], rows_v, sem).wait()
            pltpu.sync_copy(rows_v, out_hbm.at[pl.ds(off, K)])
            return carry

        lax.fori_loop(0, chunks, body, 0)

    return k(table, idx)


def _sc_segsum(dst_s, exv, z):
    mesh = plsc.VectorSubcoreMesh(core_axis_name="c", subcore_axis_name="s")
    n_vregs = DCH // LANE

    @functools.partial(
        pl.kernel,
        out_type=jax.ShapeDtypeStruct((N_OUT, 384), jnp.float32),
        mesh=mesh,
        scratch_types=[
            pltpu.VMEM((DCH,), jnp.int32),
            pltpu.VMEM((HITCAP + CB,), jnp.int32),
            pltpu.VMEM((HITCAP + CB,), jnp.int32),
            pltpu.VMEM((SUBCAP + CB,), jnp.int32),
            pltpu.VMEM((SUBCAP + CB,), jnp.int32),
            pltpu.VMEM((CB,), jnp.int32),
            pltpu.VMEM((CB, 384), jnp.float32),
            pltpu.VMEM((SUB, 384), jnp.float32),
            pltpu.SemaphoreType.DMA,
        ],
        compiler_params=pltpu.CompilerParams(needs_layout_passes=False),
    )
    def k(dst_hbm, exv_hbm, z_hbm, out_hbm,
          dstv, cbuf, dbuf, cbuf2, dbuf2, gidx, ch, accv, sem):
        cid = lax.axis_index("c")
        sid = lax.axis_index("s")
        lo = (cid * NS + sid) * STRIPE

        iota = lax.iota(jnp.int32, LANE)

        def chunk_body(dc, cnt0):
            pltpu.sync_copy(dst_hbm.at[pl.ds(dc * DCH, DCH)], dstv)

            def scan_body(vi, cnt):
                d = dstv[pl.ds(vi * LANE, LANE)]
                m = (d >= lo) & (d < lo + STRIPE)
                mi = jnp.where(m, 1, 0).astype(jnp.int32)
                pos = plsc.cumsum(mi)
                at = cnt + pos - 1
                plsc.store_scatter(cbuf, [at],
                                   dc * DCH + vi * LANE + iota,
                                   mask=m)
                plsc.store_scatter(dbuf, [at], d - lo, mask=m)
                return cnt + jnp.sum(mi)

            return lax.fori_loop(0, n_vregs, scan_body, cnt0)

        count = lax.fori_loop(0, E_PAD // DCH, chunk_body, jnp.int32(0))

        for j in range(CB // LANE):
            fill = E + j * LANE + iota
            plsc.store_scatter(cbuf, [count + j * LANE + iota], fill)
            plsc.store_scatter(dbuf, [count + j * LANE + iota],
                               jnp.zeros((LANE,), jnp.int32))

        nbig = (count + LANE - 1) // LANE

        def sub_body(q, carry):
            sublo = q * SUB
            pltpu.sync_copy(z_hbm, accv)

            def rescan(vi, c2):
                dv = dbuf[pl.ds(vi * LANE, LANE)]
                gv = cbuf[pl.ds(vi * LANE, LANE)]
                m = (dv >= sublo) & (dv < sublo + SUB)
                mi = jnp.where(m, 1, 0).astype(jnp.int32)
                pos = plsc.cumsum(mi)
                at = c2 + pos - 1
                plsc.store_scatter(cbuf2, [at], gv, mask=m)
                plsc.store_scatter(dbuf2, [at], dv - sublo, mask=m)
                return c2 + jnp.sum(mi)

            cntq = lax.fori_loop(0, nbig, rescan, jnp.int32(0))

            for j in range(CB // LANE):
                fill = E + CB + j * LANE + iota
                plsc.store_scatter(cbuf2, [cntq + j * LANE + iota], fill)
                plsc.store_scatter(dbuf2, [cntq + j * LANE + iota],
                                   jnp.zeros((LANE,), jnp.int32))

            nch = (cntq + CB - 1) // CB

            def flush_body(ci, carry2):
                off = ci * CB
                for v in range(CB // LANE):
                    gidx[pl.ds(v * LANE, LANE)] = cbuf2[pl.ds(off + v * LANE,
                                                              LANE)]
                pltpu.async_copy(exv_hbm.at[gidx], ch, sem).wait()
                for v in range(CB // LANE):
                    rv = dbuf2[pl.ds(off + v * LANE, LANE)]
                    for e in range(LANE):
                        row = rv[e]
                        for kk in range(384 // LANE):
                            plsc.addupdate(
                                accv.at[row, pl.ds(kk * LANE, LANE)],
                                ch[v * LANE + e, pl.ds(kk * LANE, LANE)])
                return carry2

            lax.fori_loop(0, nch, flush_body, 0)

            pltpu.sync_copy(accv, out_hbm.at[pl.ds(lo + sublo, SUB)])
            return carry

        lax.fori_loop(0, STRIPE // SUB, sub_body, 0)

    return k(dst_s, exv, z)


def _tc_edge(gsd, ea, W1a, W1b, W1c, b1, W2, b2, W_ihT, b_ih, W_hhT, b_hh):
    BE = 512
    grid = E_PAD // BE
    eblk = E_PAD // BE

    def body(s_ref, d_ref, ea_ref, w1a, w1b, w1c, b1r, w2, b2r, wih, bih,
             whh, bhh, out_ref):
        sm = s_ref[...]
        dm = d_ref[...]
        h = sm @ w1a[...] + dm @ w1b[...] + ea_ref[...] @ w1c[...] + b1r[...]
        h = jnp.maximum(h, 0.0)
        msg = h @ w2[...] + b2r[...]
        gi = msg @ wih[...] + bih[...]
        gh = dm @ whh[...] + bhh[...]
        r = jax.nn.sigmoid(gi[:, :D] + gh[:, :D])
        z = jax.nn.sigmoid(gi[:, D:2 * D] + gh[:, D:2 * D])
        n = jnp.tanh(gi[:, 2 * D:] + r * gh[:, 2 * D:])
        out_ref[...] = (1.0 - z) * n + z * dm

    full = lambda shape: pl.BlockSpec(shape, lambda i: (0, 0))
    return pl.pallas_call(
        body,
        grid=(grid,),
        in_specs=[
            pl.BlockSpec((BE, D), lambda i: (i, 0)),
            pl.BlockSpec((BE, D), lambda i: (i + eblk, 0)),
            pl.BlockSpec((BE, 16), lambda i: (i, 0)),
            full((D, D)), full((D, D)), full((16, D)), full((1, D)),
            full((D, D)), full((1, D)),
            full((D, 3 * D)), full((1, 3 * D)),
            full((D, 3 * D)), full((1, 3 * D)),
        ],
        out_specs=pl.BlockSpec((BE, D), lambda i: (i, 0)),
        out_shape=jax.ShapeDtypeStruct((E_PAD, D), jnp.float32),
    )(gsd, gsd, ea, W1a, W1b, W1c, b1, W2, b2, W_ihT, b_ih, W_hhT, b_hh)


def _tc_qkvs(gup, mem, w2d, Wq, bq, Wkv, bkv, Wsk, bsk):
    BN = 800
    grid = N // BN

    def body(g_ref, m_ref, w_ref, wq, bqr, wkv, bkvr, wsk, bskr,
             q_out, kv_out, sk_out):
        keep = w_ref[...] >= 0
        mem2 = jnp.where(keep, g_ref[...], m_ref[...])
        q_out[...] = mem2 @ wq[...] + bqr[...]
        kv_out[...] = mem2 @ wkv[...] + bkvr[...]
        sk_out[...] = mem2 @ wsk[...] + bskr[...]

    full = lambda shape: pl.BlockSpec(shape, lambda i: (0, 0))
    return pl.pallas_call(
        body,
        grid=(grid,),
        in_specs=[
            pl.BlockSpec((BN, D), lambda i: (i, 0)),
            pl.BlockSpec((BN, D), lambda i: (i, 0)),
            pl.BlockSpec((BN, 1), lambda i: (i, 0)),
            full((D, 256)), full((1, 256)),
            full((D, 512)), full((1, 512)),
            full((D, 256)), full((1, 256)),
        ],
        out_specs=[
            pl.BlockSpec((BN, 256), lambda i: (i, 0)),
            pl.BlockSpec((BN, 512), lambda i: (i, 0)),
            pl.BlockSpec((BN, 256), lambda i: (i, 0)),
        ],
        out_shape=[
            jax.ShapeDtypeStruct((N, 256), jnp.float32),
            jax.ShapeDtypeStruct((N, 512), jnp.float32),
            jax.ShapeDtypeStruct((N, 256), jnp.float32),
        ],
    )(gup, mem, w2d, Wq, bq, Wkv, bkv, Wsk, bsk)


def _tc_exv(q_i, kv_j):
    BE = 512
    grid = E_PAD // BE
    inv = 1.0 / (D ** 0.5)

    def body(q_ref, kv_ref, exv_out):
        i = pl.program_id(0)
        q = q_ref[...]
        kv = kv_ref[...]
        a0 = jnp.sum(q[:, :D] * kv[:, :D], axis=1, keepdims=True)
        a1 = jnp.sum(q[:, D:2 * D] * kv[:, D:2 * D], axis=1, keepdims=True)
        rows = i * BE + lax.broadcasted_iota(jnp.int32, (BE, 1), 0)
        valid = rows < E
        ex0 = jnp.where(valid, jnp.exp(a0 * inv), 0.0)
        ex1 = jnp.where(valid, jnp.exp(a1 * inv), 0.0)
        exv_out[...] = jnp.concatenate(
            [kv[:, 2 * D:3 * D] * ex0, kv[:, 3 * D:] * ex1,
             ex0, ex1, jnp.zeros((BE, 126), jnp.float32)], axis=1)

    return pl.pallas_call(
        body,
        grid=(grid,),
        in_specs=[
            pl.BlockSpec((BE, 256), lambda i: (i, 0)),
            pl.BlockSpec((BE, 512), lambda i: (i, 0)),
        ],
        out_specs=pl.BlockSpec((BE, 384), lambda i: (i, 0)),
        out_shape=jax.ShapeDtypeStruct((E_PAD, 384), jnp.float32),
    )(q_i, kv_j)


def _tc_final(acc, skip, Wc, bc):
    BN = 800
    grid = N // BN

    def body(a_ref, s_ref, wc, bcr, out_ref):
        a = a_ref[...]
        d0 = a[:, 256:257] + 1e-16
        d1 = a[:, 257:258] + 1e-16
        agg = jnp.concatenate([a[:, :D] / d0, a[:, D:256] / d1], axis=1)
        out_ref[...] = (agg + s_ref[...]) @ wc[...] + bcr[...]

    full = lambda shape: pl.BlockSpec(shape, lambda i: (0, 0))
    return pl.pallas_call(
        body,
        grid=(grid,),
        in_specs=[
            pl.BlockSpec((BN, 384), lambda i: (i, 0)),
            pl.BlockSpec((BN, 256), lambda i: (i, 0)),
            full((256, D)), full((1, D)),
        ],
        out_specs=pl.BlockSpec((BN, D), lambda i: (i, 0)),
        out_shape=jax.ShapeDtypeStruct((N, D), jnp.float32),
    )(acc, skip, Wc, bc)


def kernel(edge_index, edge_time, edge_attr, memory, W1, b1, W2, b2,
           W_ih, W_hh, b_ih, b_hh, Wq, bq, Wk, bk, Wv, bv,
           Wskip, bskip, Wc, bc):
    src = edge_index[0]
    dst = edge_index[1]

    pad = E_PAD - E
    src_g = jnp.pad(src, (0, pad))
    dst_g = jnp.pad(dst, (0, pad))
    dst_s = jnp.pad(dst, (0, pad), constant_values=-1)
    ea_pad = jnp.pad(edge_attr, ((0, pad), (0, 0)))

    idx2 = jnp.concatenate([src_g, dst_g])
    gsd = _sc_gather(memory, idx2, K=128)

    W1a, W1b, W1c = W1[:D], W1[D:2 * D], W1[2 * D:]
    upd = _tc_edge(gsd, ea_pad, W1a, W1b, W1c, b1.reshape(1, -1),
                   W2, b2.reshape(1, -1),
                   W_ih.T, b_ih.reshape(1, -1),
                   W_hh.T, b_hh.reshape(1, -1))

    eids = jnp.arange(E, dtype=jnp.int32)
    w = jnp.full((N,), -1, jnp.int32).at[dst].max(eids)
    wc_pad = jnp.pad(jnp.maximum(w, 0), (0, N_PAD - N))
    gup = _sc_gather(upd, wc_pad, K=128)

    Wkv = jnp.concatenate([Wk, Wv], axis=1)
    Q, KV, SKIP = _tc_qkvs(gup, memory, w.reshape(-1, 1),
                           Wq, bq.reshape(1, -1),
                           Wkv, jnp.concatenate([bk, bv]).reshape(1, -1),
                           Wskip, bskip.reshape(1, -1))

    q_i = _sc_gather(Q, dst_g, K=128)
    kv_j = _sc_gather(KV, src_g, K=128)

    exv = _tc_exv(q_i, kv_j)

    acc = _sc_segsum(dst_s, exv, jnp.zeros((SUB, 384), jnp.float32))

    return _tc_final(acc, SKIP, Wc, bc.reshape(1, -1))
